# baseline (speedup 1.0000x reference)
import functools

import jax
import jax.numpy as jnp
from jax import lax
from jax.experimental import pallas as pl
from jax.experimental.pallas import tpu as pltpu
from jax.experimental.pallas import tpu_sc as plsc

TOPK = 128
N = 8192
ROWS = 4096
NC = 2
NS = 16
NW = NC * NS
RPW = ROWS // NW
NV = N // 16
CAP = N
OCAP = TOPK

_INT_MAX = 2147483647
_PRED = 2.03125


def _key(bits):
    return bits ^ lax.shift_right_logical(lax.shift_right_arithmetic(bits, 31), 1)


def _make_kernel():
    mesh = plsc.VectorSubcoreMesh(
        core_axis_name="c", subcore_axis_name="s", num_cores=NC, num_subcores=NS
    )

    @functools.partial(
        pl.kernel,
        out_type=jax.ShapeDtypeStruct((ROWS, TOPK), jnp.float32),
        mesh=mesh,
        compiler_params=pltpu.CompilerParams(needs_layout_passes=False),
        scratch_types=[
            pltpu.VMEM((2 * N,), jnp.float32),
            pltpu.VMEM((3 * CAP,), jnp.float32),
            pltpu.VMEM((2 * OCAP,), jnp.float32),
            pltpu.VMEM((16,), jnp.int32),
            pltpu.VMEM((256,), jnp.int32),
            pltpu.SemaphoreType.DMA,
            pltpu.SemaphoreType.DMA,
            pltpu.SemaphoreType.DMA,
            pltpu.SemaphoreType.DMA,
        ],
    )
    def kern(x_hbm, out_hbm, rowbuf, cand, outbuf, hist, hist256,
             sem0, sem1, osem0, osem1):
        c = lax.axis_index("c")
        s = lax.axis_index("s")
        wid = s * NC + c
        r0 = wid * RPW

        iota = lax.iota(jnp.int32, 16)
        ones = jnp.ones((16,), jnp.int32)
        zeros = jnp.zeros((16,), jnp.int32)
        predv = jnp.full((16,), _PRED, jnp.float32)

        def keys_of(vec):
            return _key(lax.bitcast_convert_type(vec, jnp.int32))

        def bcast_at(x, idx_v):
            return x.at[idx_v].get(mode="promise_in_bounds")

        def scan_vec(hv, rank_v):
            hr = lax.rev(hv, (0,))
            cs = plsc.cumsum(hr)
            ge = cs >= rank_v
            p = plsc.all_reduce_ffs(ge)
            bstar_v = 15 - p
            above = bcast_at(cs - hr, p)
            return bstar_v, rank_v - above

        def extract_pred(base):
            def e_body(i, ncv1):
                v = rowbuf[pl.ds(base + i * 16, 16)]
                msk = v >= predv
                pos = ncv1 + plsc.cumsum(ones, mask=msk)
                plsc.store_scatter(cand, [pos], v, mask=msk)
                return ncv1 + plsc.all_reduce_population_count(msk)

            m1 = jnp.full((16,), -1, jnp.int32)
            return jnp.max(
                plsc.parallel_loop(0, NV, unroll=8, carry=m1)(e_body)
            ) + 1

        def hist_fallback(base):
            hist[...] = zeros

            def h_body(i, _):
                ks = keys_of(rowbuf[pl.ds(base + i * 16, 16)])
                nib = lax.shift_right_logical(ks, 28) ^ 8
                plsc.addupdate_scatter(hist, [nib], ones)
                return 0

            lax.fori_loop(0, NV, h_body, 0)
            bstar_v, _ = scan_vec(hist[...], jnp.full((16,), TOPK, jnp.int32))

            def e_body(i, ncv):
                v = rowbuf[pl.ds(base + i * 16, 16)]
                nib = lax.shift_right_logical(keys_of(v), 28) ^ 8
                msk = nib >= bstar_v
                pos = ncv + plsc.cumsum(ones, mask=msk) - 1
                plsc.store_scatter(cand, [pos], v, mask=msk)
                return ncv + plsc.all_reduce_population_count(msk)

            nc = jnp.max(lax.fori_loop(0, NV, e_body, zeros, unroll=2))
            return nc, jnp.int32(TOPK), jnp.int32(24)

        def refine(nc0, rank0, shift0):
            def lcond(st):
                shift, src, nc, rank_v = st
                return (nc > 16) & (shift >= 0)

            def lbody(st):
                shift, src, nc, rank_v = st
                sbase = src * CAP
                hist[...] = zeros
                for z in range(16):
                    hist256[pl.ds(z * 16, 16)] = zeros
                nvc = (nc + 15) >> 4
                byte_xor = jnp.where(shift == 24, jnp.int32(0x80), jnp.int32(0))

                def h_body(i, carry):
                    valid = iota + i * 16 < nc
                    kv = keys_of(cand[pl.ds(sbase + i * 16, 16)])
                    byte = (lax.shift_right_logical(kv, shift) & 255) ^ byte_xor
                    plsc.addupdate_scatter(hist256, [byte], ones, mask=valid)
                    plsc.addupdate_scatter(
                        hist, [lax.shift_right_logical(byte, 4)], ones, mask=valid
                    )
                    return carry

                plsc.parallel_loop(0, nvc, unroll=4, carry=jnp.int32(0))(h_body)
                bc_v, rank1 = scan_vec(hist[...], rank_v)
                fine = plsc.load_gather(hist256, [bc_v * 16 + iota])
                bf_v, rank_n = scan_vec(fine, rank1)
                byte_star_raw = (bc_v * 16 + bf_v) ^ byte_xor
                dsrc = jnp.where(src == 1, jnp.int32(2), jnp.int32(1))
                dbase = dsrc * CAP

                def e_body(i, nv2):
                    valid = iota + i * 16 < nc
                    vv = cand[pl.ds(sbase + i * 16, 16)]
                    kv = keys_of(vv)
                    byte = lax.shift_right_logical(kv, shift) & 255
                    msk = valid & (byte == byte_star_raw)
                    pos = nv2 + plsc.cumsum(ones, mask=msk)
                    plsc.store_scatter(cand, [dbase + pos], vv, mask=msk)
                    return nv2 + plsc.all_reduce_population_count(msk)

                m1 = jnp.full((16,), -1, jnp.int32)
                nc_n = jnp.max(
                    plsc.parallel_loop(0, nvc, unroll=4, carry=m1)(e_body)
                ) + 1
                return (shift - 8, dsrc, nc_n, rank_n)

            rank_v0 = jnp.broadcast_to(rank0, (16,))
            shift, src, nc, rank_v = lax.while_loop(
                lcond, lbody, (shift0, jnp.int32(0), nc0, rank_v0)
            )

            def sort_term(_):
                vmask = iota < nc
                ksv = keys_of(cand[pl.ds(src * CAP, 16)])
                sk, _, _ = plsc.sort_key_val(ksv, ksv, mask=vmask, descending=True)
                T_v = bcast_at(sk, rank_v - 1)
                g_v = plsc.all_reduce_population_count(vmask & (ksv > T_v))
                ntie_v = plsc.all_reduce_population_count(vmask & (ksv == T_v))
                m_v = rank_v - g_v
                simple_v = jnp.where(m_v == ntie_v, ones, zeros)
                return T_v, m_v, simple_v

            def equal_term(_):
                T_v = bcast_at(keys_of(cand[pl.ds(src * CAP, 16)]), zeros)
                simple_v = jnp.where(rank_v == nc, ones, zeros)
                return T_v, rank_v, simple_v

            T_v, m_v, simple_v = lax.cond(nc <= 16, sort_term, equal_term, 0)
            return jnp.max(simple_v), T_v, m_v

        def final_from_cand(nc0, obase, simple, T_v, m_v):
            tfv = lax.bitcast_convert_type(_key(T_v), jnp.float32)
            nvc = (nc0 + 15) >> 4

            def fin_simple(_):
                def fb(i, pv1):
                    valid = iota < nc0 - i * 16
                    v = cand[pl.ds(i * 16, 16)]
                    msk = valid & (v >= tfv)
                    pos = pv1 + plsc.cumsum(ones, mask=msk)
                    plsc.store_scatter(outbuf, [obase + pos], v, mask=msk)
                    return pv1 + plsc.all_reduce_population_count(msk)

                m1 = jnp.full((16,), -1, jnp.int32)
                plsc.parallel_loop(0, nvc, unroll=4, carry=m1)(fb)
                return 0

            def fin_general(_):
                def fb(i, carry):
                    pv, tv = carry
                    valid = iota < nc0 - i * 16
                    v = cand[pl.ds(i * 16, 16)]
                    gt = valid & (v > tfv)
                    tie = valid & (v == tfv)
                    tcs = plsc.cumsum(ones, mask=tie)
                    keep = gt | (tie & (tv + tcs - 1 < m_v))
                    pos = pv + plsc.cumsum(ones, mask=keep) - 1
                    plsc.store_scatter(outbuf, [obase + pos], v, mask=keep)
                    return (pv + plsc.all_reduce_population_count(keep),
                            tv + plsc.all_reduce_population_count(tie))

                lax.fori_loop(0, nvc, fb, (zeros, zeros))
                return 0

            lax.cond(simple == 1, fin_simple, fin_general, 0)

        def do_row(j, base, sem, osem, obase):
            row = r0 + j
            pltpu.make_async_copy(
                x_hbm.at[row], rowbuf.at[pl.ds(base, N)], sem
            ).wait()

            nc_p = extract_pred(base)
            nc, rank, shift0 = lax.cond(
                nc_p >= TOPK,
                lambda _: (nc_p, jnp.int32(TOPK), jnp.int32(24)),
                lambda _: hist_fallback(base),
                0,
            )

            @pl.when(j + 2 < RPW)
            def _():
                pltpu.async_copy(
                    x_hbm.at[row + 2], rowbuf.at[pl.ds(base, N)], sem
                )

            simple, T_v, m_v = refine(nc, rank, shift0)

            @pl.when(j >= 2)
            def _():
                pltpu.make_async_copy(
                    outbuf.at[pl.ds(obase, TOPK)], out_hbm.at[row - 2], osem
                ).wait()

            final_from_cand(nc, obase, simple, T_v, m_v)
            pltpu.async_copy(
                outbuf.at[pl.ds(obase, TOPK)], out_hbm.at[row], osem
            )

        pltpu.async_copy(x_hbm.at[r0], rowbuf.at[pl.ds(0, N)], sem0)
        pltpu.async_copy(x_hbm.at[r0 + 1], rowbuf.at[pl.ds(N, N)], sem1)

        def pair_body(p, _):
            do_row(2 * p, 0, sem0, osem0, 0)
            do_row(2 * p + 1, N, sem1, osem1, OCAP)
            return 0

        lax.fori_loop(0, RPW // 2, pair_body, 0)

        pltpu.make_async_copy(
            outbuf.at[pl.ds(0, TOPK)], out_hbm.at[r0 + RPW - 2], osem0
        ).wait()
        pltpu.make_async_copy(
            outbuf.at[pl.ds(OCAP, TOPK)], out_hbm.at[r0 + RPW - 1], osem1
        ).wait()

    return kern


_kern = _make_kernel()


@jax.jit
def _kmax(x):
    return _kern(x)


def kernel(inputs):
    x = inputs.reshape(ROWS, N)
    out = _kmax(x)
    return out.reshape(inputs.shape[0], inputs.shape[1], TOPK)


# speedup vs baseline: 1.0596x; 1.0596x over previous
import functools

import jax
import jax.numpy as jnp
from jax import lax
from jax.experimental import pallas as pl
from jax.experimental.pallas import tpu as pltpu
from jax.experimental.pallas import tpu_sc as plsc

TOPK = 128
N = 8192
ROWS = 4096
NC = 2
NS = 16
NW = NC * NS
RPW = ROWS // NW
NV = N // 16
CAP = N
OCAP = TOPK

_INT_MAX = 2147483647
_PRED = 2.03125


def _key(bits):
    return bits ^ lax.shift_right_logical(lax.shift_right_arithmetic(bits, 31), 1)


def _make_kernel():
    mesh = plsc.VectorSubcoreMesh(
        core_axis_name="c", subcore_axis_name="s", num_cores=NC, num_subcores=NS
    )

    @functools.partial(
        pl.kernel,
        out_type=jax.ShapeDtypeStruct((ROWS, TOPK), jnp.float32),
        mesh=mesh,
        compiler_params=pltpu.CompilerParams(needs_layout_passes=False),
        scratch_types=[
            pltpu.VMEM((2 * N,), jnp.float32),
            pltpu.VMEM((3 * CAP,), jnp.float32),
            pltpu.VMEM((2 * OCAP,), jnp.float32),
            pltpu.VMEM((16,), jnp.int32),
            pltpu.VMEM((256,), jnp.int32),
            pltpu.SemaphoreType.DMA,
            pltpu.SemaphoreType.DMA,
            pltpu.SemaphoreType.DMA,
            pltpu.SemaphoreType.DMA,
        ],
    )
    def kern(x_hbm, out_hbm, rowbuf, cand, outbuf, hist, hist256,
             sem0, sem1, osem0, osem1):
        c = lax.axis_index("c")
        s = lax.axis_index("s")
        wid = s * NC + c
        r0 = wid * RPW

        iota = lax.iota(jnp.int32, 16)
        ones = jnp.ones((16,), jnp.int32)
        zeros = jnp.zeros((16,), jnp.int32)
        predv = jnp.full((16,), _PRED, jnp.float32)

        def keys_of(vec):
            return _key(lax.bitcast_convert_type(vec, jnp.int32))

        def bcast_at(x, idx_v):
            return x.at[idx_v].get(mode="promise_in_bounds")

        def scan_vec(hv, rank_v):
            hr = lax.rev(hv, (0,))
            cs = plsc.cumsum(hr)
            ge = cs >= rank_v
            p = plsc.all_reduce_ffs(ge)
            bstar_v = 15 - p
            above = bcast_at(cs - hr, p)
            return bstar_v, rank_v - above

        def extract_pred(base):
            def e_body(i, ncv1):
                v = rowbuf[pl.ds(base + i * 16, 16)]
                msk = v >= predv
                pos = ncv1 + plsc.cumsum(ones, mask=msk)
                plsc.store_scatter(cand, [pos], v, mask=msk)
                return ncv1 + plsc.all_reduce_population_count(msk)

            m1 = jnp.full((16,), -1, jnp.int32)
            return jnp.max(
                plsc.parallel_loop(0, NV, unroll=8, carry=m1)(e_body)
            ) + 1

        def hist_fallback(base):
            hist[...] = zeros

            def h_body(i, _):
                ks = keys_of(rowbuf[pl.ds(base + i * 16, 16)])
                nib = lax.shift_right_logical(ks, 28) ^ 8
                plsc.addupdate_scatter(hist, [nib], ones)
                return 0

            lax.fori_loop(0, NV, h_body, 0)
            bstar_v, _ = scan_vec(hist[...], jnp.full((16,), TOPK, jnp.int32))

            def e_body(i, ncv):
                v = rowbuf[pl.ds(base + i * 16, 16)]
                nib = lax.shift_right_logical(keys_of(v), 28) ^ 8
                msk = nib >= bstar_v
                pos = ncv + plsc.cumsum(ones, mask=msk) - 1
                plsc.store_scatter(cand, [pos], v, mask=msk)
                return ncv + plsc.all_reduce_population_count(msk)

            nc = jnp.max(lax.fori_loop(0, NV, e_body, zeros, unroll=2))
            return nc, jnp.int32(TOPK), jnp.int32(24)

        def refine(nc0, rank0, shift0):
            def lcond(st):
                shift, src, nc, rank_v = st
                return (nc > 16) & (shift >= 0)

            def lbody(st):
                shift, src, nc, rank_v = st
                sbase = src * CAP
                hist[...] = zeros
                for z in range(16):
                    hist256[pl.ds(z * 16, 16)] = zeros
                nvc = (nc + 15) >> 4
                byte_xor = jnp.where(shift == 24, jnp.int32(0x80), jnp.int32(0))

                def h_body(i, carry):
                    valid = iota + i * 16 < nc
                    kv = keys_of(cand[pl.ds(sbase + i * 16, 16)])
                    byte = (lax.shift_right_logical(kv, shift) & 255) ^ byte_xor
                    plsc.addupdate_scatter(hist256, [byte], ones, mask=valid)
                    plsc.addupdate_scatter(
                        hist, [lax.shift_right_logical(byte, 4)], ones, mask=valid
                    )
                    return carry

                plsc.parallel_loop(0, nvc, unroll=2, carry=jnp.int32(0))(h_body)
                bc_v, rank1 = scan_vec(hist[...], rank_v)
                fine = plsc.load_gather(hist256, [bc_v * 16 + iota])
                bf_v, rank_n = scan_vec(fine, rank1)
                byte_star_raw = (bc_v * 16 + bf_v) ^ byte_xor
                dsrc = jnp.where(src == 1, jnp.int32(2), jnp.int32(1))
                dbase = dsrc * CAP

                def e_body(i, nv2):
                    valid = iota + i * 16 < nc
                    vv = cand[pl.ds(sbase + i * 16, 16)]
                    kv = keys_of(vv)
                    byte = lax.shift_right_logical(kv, shift) & 255
                    msk = valid & (byte == byte_star_raw)
                    pos = nv2 + plsc.cumsum(ones, mask=msk)
                    plsc.store_scatter(cand, [dbase + pos], vv, mask=msk)
                    return nv2 + plsc.all_reduce_population_count(msk)

                m1 = jnp.full((16,), -1, jnp.int32)
                nc_n = jnp.max(
                    plsc.parallel_loop(0, nvc, unroll=2, carry=m1)(e_body)
                ) + 1
                return (shift - 8, dsrc, nc_n, rank_n)

            rank_v0 = jnp.broadcast_to(rank0, (16,))
            shift, src, nc, rank_v = lax.while_loop(
                lcond, lbody, (shift0, jnp.int32(0), nc0, rank_v0)
            )

            def sort_term(_):
                vmask = iota < nc
                ksv = keys_of(cand[pl.ds(src * CAP, 16)])
                sk, _, _ = plsc.sort_key_val(ksv, ksv, mask=vmask, descending=True)
                T_v = bcast_at(sk, rank_v - 1)
                g_v = plsc.all_reduce_population_count(vmask & (ksv > T_v))
                ntie_v = plsc.all_reduce_population_count(vmask & (ksv == T_v))
                m_v = rank_v - g_v
                simple_v = jnp.where(m_v == ntie_v, ones, zeros)
                return T_v, m_v, simple_v

            def equal_term(_):
                T_v = bcast_at(keys_of(cand[pl.ds(src * CAP, 16)]), zeros)
                simple_v = jnp.where(rank_v == nc, ones, zeros)
                return T_v, rank_v, simple_v

            T_v, m_v, simple_v = lax.cond(nc <= 16, sort_term, equal_term, 0)
            return jnp.max(simple_v), T_v, m_v

        def final_from_cand(nc0, obase, simple, T_v, m_v):
            tfv = lax.bitcast_convert_type(_key(T_v), jnp.float32)
            nvc = (nc0 + 15) >> 4

            def fin_simple(_):
                def fb(i, pv1):
                    valid = iota < nc0 - i * 16
                    v = cand[pl.ds(i * 16, 16)]
                    msk = valid & (v >= tfv)
                    pos = pv1 + plsc.cumsum(ones, mask=msk)
                    plsc.store_scatter(outbuf, [obase + pos], v, mask=msk)
                    return pv1 + plsc.all_reduce_population_count(msk)

                m1 = jnp.full((16,), -1, jnp.int32)
                plsc.parallel_loop(0, nvc, unroll=2, carry=m1)(fb)
                return 0

            def fin_general(_):
                def fb(i, carry):
                    pv, tv = carry
                    valid = iota < nc0 - i * 16
                    v = cand[pl.ds(i * 16, 16)]
                    gt = valid & (v > tfv)
                    tie = valid & (v == tfv)
                    tcs = plsc.cumsum(ones, mask=tie)
                    keep = gt | (tie & (tv + tcs - 1 < m_v))
                    pos = pv + plsc.cumsum(ones, mask=keep) - 1
                    plsc.store_scatter(outbuf, [obase + pos], v, mask=keep)
                    return (pv + plsc.all_reduce_population_count(keep),
                            tv + plsc.all_reduce_population_count(tie))

                lax.fori_loop(0, nvc, fb, (zeros, zeros))
                return 0

            lax.cond(simple == 1, fin_simple, fin_general, 0)

        def do_row(j, base, sem, osem, obase):
            row = r0 + j
            pltpu.make_async_copy(
                x_hbm.at[row], rowbuf.at[pl.ds(base, N)], sem
            ).wait()

            nc_p = extract_pred(base)
            nc, rank, shift0 = lax.cond(
                nc_p >= TOPK,
                lambda _: (nc_p, jnp.int32(TOPK), jnp.int32(24)),
                lambda _: hist_fallback(base),
                0,
            )

            @pl.when(j + 2 < RPW)
            def _():
                pltpu.async_copy(
                    x_hbm.at[row + 2], rowbuf.at[pl.ds(base, N)], sem
                )

            simple, T_v, m_v = refine(nc, rank, shift0)

            @pl.when(j >= 2)
            def _():
                pltpu.make_async_copy(
                    outbuf.at[pl.ds(obase, TOPK)], out_hbm.at[row - 2], osem
                ).wait()

            final_from_cand(nc, obase, simple, T_v, m_v)
            pltpu.async_copy(
                outbuf.at[pl.ds(obase, TOPK)], out_hbm.at[row], osem
            )

        pltpu.async_copy(x_hbm.at[r0], rowbuf.at[pl.ds(0, N)], sem0)
        pltpu.async_copy(x_hbm.at[r0 + 1], rowbuf.at[pl.ds(N, N)], sem1)

        def pair_body(p, _):
            do_row(2 * p, 0, sem0, osem0, 0)
            do_row(2 * p + 1, N, sem1, osem1, OCAP)
            return 0

        lax.fori_loop(0, RPW // 2, pair_body, 0)

        pltpu.make_async_copy(
            outbuf.at[pl.ds(0, TOPK)], out_hbm.at[r0 + RPW - 2], osem0
        ).wait()
        pltpu.make_async_copy(
            outbuf.at[pl.ds(OCAP, TOPK)], out_hbm.at[r0 + RPW - 1], osem1
        ).wait()

    return kern


_kern = _make_kernel()


@jax.jit
def _kmax(x):
    return _kern(x)


def kernel(inputs):
    x = inputs.reshape(ROWS, N)
    out = _kmax(x)
    return out.reshape(inputs.shape[0], inputs.shape[1], TOPK)


# speedup vs baseline: 1.0603x; 1.0007x over previous
import functools

import jax
import jax.numpy as jnp
from jax import lax
from jax.experimental import pallas as pl
from jax.experimental.pallas import tpu as pltpu
from jax.experimental.pallas import tpu_sc as plsc

TOPK = 128
N = 8192
ROWS = 4096
NC = 2
NS = 16
NW = NC * NS
RPW = ROWS // NW
NV = N // 16
CAP = N
OCAP = TOPK

_PRED = 2.03125


def _key(bits):
    return bits ^ lax.shift_right_logical(lax.shift_right_arithmetic(bits, 31), 1)


def _make_kernel():
    mesh = plsc.VectorSubcoreMesh(
        core_axis_name="c", subcore_axis_name="s", num_cores=NC, num_subcores=NS
    )

    @functools.partial(
        pl.kernel,
        out_type=jax.ShapeDtypeStruct((ROWS, TOPK), jnp.float32),
        mesh=mesh,
        compiler_params=pltpu.CompilerParams(needs_layout_passes=False),
        scratch_types=[
            pltpu.VMEM((2 * N,), jnp.float32),
            pltpu.VMEM((3 * CAP,), jnp.float32),
            pltpu.VMEM((2 * OCAP,), jnp.float32),
            pltpu.VMEM((16,), jnp.int32),
            pltpu.VMEM((256,), jnp.int32),
            pltpu.SemaphoreType.DMA,
            pltpu.SemaphoreType.DMA,
            pltpu.SemaphoreType.DMA,
            pltpu.SemaphoreType.DMA,
        ],
    )
    def kern(x_hbm, out_hbm, rowbuf, cand, outbuf, hist, hist256,
             sem0, sem1, osem0, osem1):
        c = lax.axis_index("c")
        s = lax.axis_index("s")
        wid = s * NC + c
        r0 = wid * RPW

        iota = lax.iota(jnp.int32, 16)
        ones = jnp.ones((16,), jnp.int32)
        zeros = jnp.zeros((16,), jnp.int32)
        predv = jnp.full((16,), _PRED, jnp.float32)

        def keys_of(vec):
            return _key(lax.bitcast_convert_type(vec, jnp.int32))

        def bcast_at(x, idx_v):
            return x.at[idx_v].get(mode="promise_in_bounds")

        def scan_vec(hv, rank_v):
            hr = lax.rev(hv, (0,))
            cs = plsc.cumsum(hr)
            ge = cs >= rank_v
            p = plsc.all_reduce_ffs(ge)
            bstar_v = 15 - p
            above = bcast_at(cs - hr, p)
            return bstar_v, rank_v - above

        def extract_pred(base):
            def e_body(i, ncv1):
                v = rowbuf[pl.ds(base + i * 16, 16)]
                msk = v >= predv
                pos = ncv1 + plsc.cumsum(ones, mask=msk)
                plsc.store_scatter(cand, [pos], v, mask=msk)
                return ncv1 + plsc.all_reduce_population_count(msk)

            m1 = jnp.full((16,), -1, jnp.int32)
            return jnp.max(
                plsc.parallel_loop(0, NV, unroll=8, carry=m1)(e_body)
            ) + 1

        def hist_fallback(base):
            hist[...] = zeros

            def h_body(i, _):
                ks = keys_of(rowbuf[pl.ds(base + i * 16, 16)])
                nib = lax.shift_right_logical(ks, 28) ^ 8
                plsc.addupdate_scatter(hist, [nib], ones)
                return 0

            lax.fori_loop(0, NV, h_body, 0)
            bstar_v, _ = scan_vec(hist[...], jnp.full((16,), TOPK, jnp.int32))

            def e_body(i, ncv):
                v = rowbuf[pl.ds(base + i * 16, 16)]
                nib = lax.shift_right_logical(keys_of(v), 28) ^ 8
                msk = nib >= bstar_v
                pos = ncv + plsc.cumsum(ones, mask=msk) - 1
                plsc.store_scatter(cand, [pos], v, mask=msk)
                return ncv + plsc.all_reduce_population_count(msk)

            nc = jnp.max(lax.fori_loop(0, NV, e_body, zeros, unroll=2))
            return nc, jnp.int32(TOPK), jnp.int32(24)

        def refine(nc0, rank0, shift0):
            def lcond(st):
                shift, src, nc, rank_v = st
                return (nc > 16) & (shift >= 0)

            def lbody(st):
                shift, src, nc, rank_v = st
                sbase = src * CAP
                hist[...] = zeros
                for z in range(16):
                    hist256[pl.ds(z * 16, 16)] = zeros
                nvc = (nc + 15) >> 4
                byte_xor = jnp.where(shift == 24, jnp.int32(0x80), jnp.int32(0))

                def h_body(i, carry):
                    valid = iota + i * 16 < nc
                    kv = keys_of(cand[pl.ds(sbase + i * 16, 16)])
                    byte = (lax.shift_right_logical(kv, shift) & 255) ^ byte_xor
                    plsc.addupdate_scatter(hist256, [byte], ones, mask=valid)
                    plsc.addupdate_scatter(
                        hist, [lax.shift_right_logical(byte, 4)], ones, mask=valid
                    )
                    return carry

                plsc.parallel_loop(0, nvc, unroll=2, carry=jnp.int32(0))(h_body)
                bc_v, rank1 = scan_vec(hist[...], rank_v)
                fine = plsc.load_gather(hist256, [bc_v * 16 + iota])
                bf_v, rank_n = scan_vec(fine, rank1)
                byte_star_raw = (bc_v * 16 + bf_v) ^ byte_xor
                dsrc = jnp.where(src == 1, jnp.int32(2), jnp.int32(1))
                dbase = dsrc * CAP

                def e_body(i, nv2):
                    valid = iota + i * 16 < nc
                    vv = cand[pl.ds(sbase + i * 16, 16)]
                    kv = keys_of(vv)
                    byte = lax.shift_right_logical(kv, shift) & 255
                    msk = valid & (byte == byte_star_raw)
                    pos = nv2 + plsc.cumsum(ones, mask=msk)
                    plsc.store_scatter(cand, [dbase + pos], vv, mask=msk)
                    return nv2 + plsc.all_reduce_population_count(msk)

                m1 = jnp.full((16,), -1, jnp.int32)
                nc_n = jnp.max(
                    plsc.parallel_loop(0, nvc, unroll=2, carry=m1)(e_body)
                ) + 1
                return (shift - 8, dsrc, nc_n, rank_n)

            rank_v0 = jnp.broadcast_to(rank0, (16,))
            shift, src, nc, rank_v = lax.while_loop(
                lcond, lbody, (shift0, jnp.int32(0), nc0, rank_v0)
            )

            def sort_term(_):
                vmask = iota < nc
                ksv = keys_of(cand[pl.ds(src * CAP, 16)])
                sk, _, _ = plsc.sort_key_val(ksv, ksv, mask=vmask, descending=True)
                T_v = bcast_at(sk, rank_v - 1)
                g_v = plsc.all_reduce_population_count(vmask & (ksv > T_v))
                ntie_v = plsc.all_reduce_population_count(vmask & (ksv == T_v))
                m_v = rank_v - g_v
                simple_v = jnp.where(m_v == ntie_v, ones, zeros)
                return T_v, m_v, simple_v

            def equal_term(_):
                T_v = bcast_at(keys_of(cand[pl.ds(src * CAP, 16)]), zeros)
                simple_v = jnp.where(rank_v == nc, ones, zeros)
                return T_v, rank_v, simple_v

            T_v, m_v, simple_v = lax.cond(nc <= 16, sort_term, equal_term, 0)
            return jnp.max(simple_v), T_v, m_v

        def final_from_cand(nc0, obase, simple, T_v, m_v):
            tfv = lax.bitcast_convert_type(_key(T_v), jnp.float32)
            nvc = (nc0 + 15) >> 4

            def fin_simple(_):
                def fb(i, pv1):
                    valid = iota < nc0 - i * 16
                    v = cand[pl.ds(i * 16, 16)]
                    msk = valid & (v >= tfv)
                    pos = pv1 + plsc.cumsum(ones, mask=msk)
                    plsc.store_scatter(outbuf, [obase + pos], v, mask=msk)
                    return pv1 + plsc.all_reduce_population_count(msk)

                m1 = jnp.full((16,), -1, jnp.int32)
                plsc.parallel_loop(0, nvc, unroll=2, carry=m1)(fb)
                return 0

            def fin_general(_):
                def fb(i, carry):
                    pv, tv = carry
                    valid = iota < nc0 - i * 16
                    v = cand[pl.ds(i * 16, 16)]
                    gt = valid & (v > tfv)
                    tie = valid & (v == tfv)
                    tcs = plsc.cumsum(ones, mask=tie)
                    keep = gt | (tie & (tv + tcs - 1 < m_v))
                    pos = pv + plsc.cumsum(ones, mask=keep) - 1
                    plsc.store_scatter(outbuf, [obase + pos], v, mask=keep)
                    return (pv + plsc.all_reduce_population_count(keep),
                            tv + plsc.all_reduce_population_count(tie))

                lax.fori_loop(0, nvc, fb, (zeros, zeros))
                return 0

            lax.cond(simple == 1, fin_simple, fin_general, 0)

        def do_row(j, base, sem, osem, obase):
            row = r0 + j
            pltpu.make_async_copy(
                x_hbm.at[row], rowbuf.at[pl.ds(base, N)], sem
            ).wait()

            nc_p = extract_pred(base)
            nc, rank, shift0 = lax.cond(
                nc_p >= TOPK,
                lambda _: (nc_p, jnp.int32(TOPK), jnp.int32(24)),
                lambda _: hist_fallback(base),
                0,
            )

            @pl.when(j + 2 < RPW)
            def _():
                pltpu.async_copy(
                    x_hbm.at[row + 2], rowbuf.at[pl.ds(base, N)], sem
                )

            simple, T_v, m_v = refine(nc, rank, shift0)

            @pl.when(j >= 2)
            def _():
                pltpu.make_async_copy(
                    outbuf.at[pl.ds(obase, TOPK)], out_hbm.at[row - 2], osem
                ).wait()

            final_from_cand(nc, obase, simple, T_v, m_v)
            pltpu.async_copy(
                outbuf.at[pl.ds(obase, TOPK)], out_hbm.at[row], osem
            )

        pltpu.async_copy(x_hbm.at[r0], rowbuf.at[pl.ds(0, N)], sem0)
        pltpu.async_copy(x_hbm.at[r0 + 1], rowbuf.at[pl.ds(N, N)], sem1)

        def pair_body(p, _):
            do_row(2 * p, 0, sem0, osem0, 0)
            do_row(2 * p + 1, N, sem1, osem1, OCAP)
            return 0

        lax.fori_loop(0, RPW // 2, pair_body, 0)

        pltpu.make_async_copy(
            outbuf.at[pl.ds(0, TOPK)], out_hbm.at[r0 + RPW - 2], osem0
        ).wait()
        pltpu.make_async_copy(
            outbuf.at[pl.ds(OCAP, TOPK)], out_hbm.at[r0 + RPW - 1], osem1
        ).wait()

    return kern


_kern = _make_kernel()


@jax.jit
def _kmax(x):
    return _kern(x)


def kernel(inputs):
    x = inputs.reshape(ROWS, N)
    out = _kmax(x)
    return out.reshape(inputs.shape[0], inputs.shape[1], TOPK)
